# baseline (speedup 1.0000x reference)
import functools

import jax
import jax.numpy as jnp
from jax import lax
from jax.experimental import pallas as pl
from jax.experimental.pallas import tpu as pltpu
from jax.experimental.pallas import tpu_sc as plsc

N = 320000
D = 128
NUM_SEGMENTS = 10000

NC = 2
NS = 16
NW = NC * NS
ROWS_PER_W = N // NW
CHUNK = 80
NCHUNK = ROWS_PER_W // CHUNK
NBUF = 3
NOUTER = (NCHUNK + NBUF - 1) // NBUF
SEG_PAD = 10240
SEG_PER_T = SEG_PAD // NS
STAGE = 64
NSTAGE = SEG_PER_T // STAGE


def _sums_body(x_hbm, b_hbm, sums_out,
               xb0, xb1, xb2, ib0, ib1, ib2, zbuf, acc,
               sx0, sx1, sx2, si0, si1, si2, ss0, ss1, ss2):
    c = lax.axis_index("c")
    s = lax.axis_index("s")
    wid = c * NS + s
    base0 = wid * ROWS_PER_W

    xbs = (xb0, xb1, xb2)
    ibs = (ib0, ib1, ib2)
    sxs = (sx0, sx1, sx2)
    sis = (si0, si1, si2)
    sss = (ss0, ss1, ss2)

    def dma_start(j, k):
        base = base0 + j * CHUNK
        pltpu.async_copy(x_hbm.at[pl.ds(base, CHUNK)], xbs[k], sxs[k])
        pltpu.async_copy(b_hbm.at[pl.ds(base, CHUNK)], ibs[k], sis[k])

    def dma_wait(j, k):
        base = base0 + j * CHUNK
        pltpu.make_async_copy(x_hbm.at[pl.ds(base, CHUNK)], xbs[k], sxs[k]).wait()
        pltpu.make_async_copy(b_hbm.at[pl.ds(base, CHUNK)], ibs[k], sis[k]).wait()

    def scat_start(k):
        pltpu.async_copy(xbs[k], acc.at[ibs[k]], sss[k], add=True)

    def scat_wait(k):
        pltpu.make_async_copy(xbs[k], acc.at[ibs[k]], sss[k]).wait()

    dma_start(0, 0)
    dma_start(1, 1)

    zero16 = jnp.zeros((16,), dtype=jnp.float32)

    def fill_zrow(i, _):
        def fill_zcol(j, _):
            zbuf[i, pl.ds(j * 16, 16)] = zero16
            return 0
        lax.fori_loop(0, D // 16, fill_zcol, 0)
        return 0
    lax.fori_loop(0, STAGE, fill_zrow, 0)

    def zero_slice(t, _):
        pltpu.sync_copy(zbuf, acc.at[pl.ds(s * SEG_PER_T + t * STAGE, STAGE)])
        return 0
    lax.fori_loop(0, NSTAGE, zero_slice, 0)
    plsc.subcore_barrier()

    def outer(i, _):
        for k in range(NBUF):
            j = i * NBUF + k
            kp = (k - 1) % NBUF

            def step():
                if k == 0:
                    @pl.when(i >= 1)
                    def _():
                        scat_wait(kp)
                else:
                    scat_wait(kp)

                @pl.when(j + NBUF - 1 < NCHUNK)
                def _():
                    dma_start(j + NBUF - 1, kp)

                dma_wait(j, k)
                scat_start(k)

            if (NOUTER - 1) * NBUF + k >= NCHUNK:
                @pl.when(j < NCHUNK)
                def _():
                    step()
            else:
                step()
        return 0
    lax.fori_loop(0, NOUTER, outer, 0)
    scat_wait((NCHUNK - 1) % NBUF)

    plsc.subcore_barrier()

    out_base = c * SEG_PAD + s * SEG_PER_T

    def stage_step(t, _):
        src = s * SEG_PER_T + t * STAGE
        pltpu.sync_copy(acc.at[pl.ds(src, STAGE)], zbuf)
        pltpu.sync_copy(zbuf, sums_out.at[pl.ds(out_base + t * STAGE, STAGE)])
        return 0
    lax.fori_loop(0, NSTAGE, stage_step, 0)


_sc_sums = functools.partial(
    pl.kernel,
    out_type=jax.ShapeDtypeStruct((NC * SEG_PAD, D), jnp.float32),
    mesh=plsc.VectorSubcoreMesh(core_axis_name="c", subcore_axis_name="s"),
    scratch_types=[
        pltpu.VMEM((CHUNK, D), jnp.float32),
        pltpu.VMEM((CHUNK, D), jnp.float32),
        pltpu.VMEM((CHUNK, D), jnp.float32),
        pltpu.VMEM((CHUNK,), jnp.int32),
        pltpu.VMEM((CHUNK,), jnp.int32),
        pltpu.VMEM((CHUNK,), jnp.int32),
        pltpu.VMEM((STAGE, D), jnp.float32),
        pltpu.VMEM_SHARED((SEG_PAD, D), jnp.float32),
        pltpu.SemaphoreType.DMA,
        pltpu.SemaphoreType.DMA,
        pltpu.SemaphoreType.DMA,
        pltpu.SemaphoreType.DMA,
        pltpu.SemaphoreType.DMA,
        pltpu.SemaphoreType.DMA,
        pltpu.SemaphoreType.DMA,
        pltpu.SemaphoreType.DMA,
        pltpu.SemaphoreType.DMA,
    ],
)(_sums_body)


BS = 1024
HK = 2000


def _hist_body(b_ref, h_ref):
    i = pl.program_id(0)

    @pl.when(i == 0)
    def _():
        h_ref[...] = jnp.zeros((D, D), jnp.float32)

    vals = b_ref[...]
    lanes = lax.broadcasted_iota(jnp.int32, (1, D), 1)
    coarse = vals // jnp.int32(D)
    fine = vals - coarse * jnp.int32(D)
    a = (coarse == lanes).astype(jnp.bfloat16)
    b = (fine == lanes).astype(jnp.bfloat16)
    h_ref[...] += lax.dot_general(
        a, b, (((0,), (0,)), ((), ())),
        preferred_element_type=jnp.float32)


def _combine_body(s_ref, c_ref, o_ref):
    sums = s_ref[0] + s_ref[1]
    cnt = c_ref[...]
    o_ref[...] = sums / jnp.maximum(cnt, 1.0)


def kernel(x, batch):
    hist = pl.pallas_call(
        _hist_body,
        out_shape=jax.ShapeDtypeStruct((D, D), jnp.float32),
        grid=(N // HK,),
        in_specs=[pl.BlockSpec((HK, 1), lambda i: (i, 0))],
        out_specs=pl.BlockSpec((D, D), lambda i: (0, 0)),
    )(batch.reshape(N, 1))
    sums = _sc_sums(x, batch)
    sums = sums.reshape(NC, SEG_PAD, D)
    cnt_col = hist.reshape(D * D, 1)
    out = pl.pallas_call(
        _combine_body,
        out_shape=jax.ShapeDtypeStruct((NUM_SEGMENTS, D), jnp.float32),
        grid=(SEG_PAD // BS,),
        in_specs=[
            pl.BlockSpec((NC, BS, D), lambda i: (0, i, 0)),
            pl.BlockSpec((BS, 1), lambda i: (i, 0)),
        ],
        out_specs=pl.BlockSpec((BS, D), lambda i: (i, 0)),
    )(sums, cnt_col)
    return out


# speedup vs baseline: 1.8178x; 1.8178x over previous
import functools

import jax
import jax.numpy as jnp
import numpy as np
from jax import lax
from jax.experimental import pallas as pl
from jax.experimental.pallas import tpu as pltpu
from jax.experimental.pallas import tpu_sc as plsc

N = 320000
D = 128
NUM_SEGMENTS = 10000

NC = 2
NS = 16
NW = NC * NS
ROWS_PER_W = N // NW
CHUNK = 80
NCHUNK = ROWS_PER_W // CHUNK
NBUF = 3
NOUTER = (NCHUNK + NBUF - 1) // NBUF
SEG_PAD = 10240
SEG_PER_T = SEG_PAD // NS
STAGE = 32
NSTAGE = SEG_PER_T // STAGE
CR = SEG_PAD // 16
HR = SEG_PAD // D


def _sums_body(x_hbm, b_hbm, eye_hbm, sums_out, cnts_out,
               xb0, xb1, xb2, ib0, ib1, ib2, zbuf, cnt1d, ohtab, acc,
               sx0, sx1, sx2, si0, si1, si2, ss0, ss1, ss2):
    c = lax.axis_index("c")
    s = lax.axis_index("s")
    wid = c * NS + s
    base0 = wid * ROWS_PER_W

    xbs = (xb0, xb1, xb2)
    ibs = (ib0, ib1, ib2)
    sxs = (sx0, sx1, sx2)
    sis = (si0, si1, si2)
    sss = (ss0, ss1, ss2)

    def dma_start(j, k):
        base = base0 + j * CHUNK
        pltpu.async_copy(x_hbm.at[pl.ds(base, CHUNK)], xbs[k], sxs[k])
        pltpu.async_copy(b_hbm.at[pl.ds(base, CHUNK)], ibs[k], sis[k])

    def dma_wait(j, k):
        base = base0 + j * CHUNK
        pltpu.make_async_copy(x_hbm.at[pl.ds(base, CHUNK)], xbs[k], sxs[k]).wait()
        pltpu.make_async_copy(b_hbm.at[pl.ds(base, CHUNK)], ibs[k], sis[k]).wait()

    def scat_start(k):
        pltpu.async_copy(xbs[k], acc.at[ibs[k]], sss[k], add=True)

    def scat_wait(k):
        pltpu.make_async_copy(xbs[k], acc.at[ibs[k]], sss[k]).wait()

    dma_start(0, 0)
    dma_start(1, 1)

    zero16 = jnp.zeros((16,), dtype=jnp.float32)
    iota16 = lax.iota(jnp.int32, 16)

    def fill_zrow(i, _):
        def fill_zcol(j, _):
            zbuf[i, pl.ds(j * 16, 16)] = zero16
            return 0
        lax.fori_loop(0, D // 16, fill_zcol, 0)
        return 0
    lax.fori_loop(0, STAGE, fill_zrow, 0)

    def fill_crow(i, _):
        cnt1d[pl.ds(i * 16, 16)] = zero16
        return 0
    lax.fori_loop(0, CR, fill_crow, 0)

    pltpu.sync_copy(eye_hbm, ohtab)

    def zero_slice(t, _):
        pltpu.sync_copy(zbuf, acc.at[pl.ds(s * SEG_PER_T + t * STAGE, STAGE)])
        return 0
    lax.fori_loop(0, NSTAGE, zero_slice, 0)
    plsc.subcore_barrier()

    def count_rows(k):
        for kb in range(CHUNK // 16):
            idxv = ibs[k][pl.ds(kb * 16, 16)]
            for m in range(16):
                v = idxv[m]
                cl = lax.rem(v, jnp.int32(16))
                off = v - cl
                oh = ohtab[pl.ds(cl * 16, 16)]
                cnt1d[pl.ds(off, 16)] = cnt1d[pl.ds(off, 16)] + oh

    def outer(i, _):
        for k in range(NBUF):
            j = i * NBUF + k
            kp = (k - 1) % NBUF

            def step():
                if k == 0:
                    @pl.when(i >= 1)
                    def _():
                        scat_wait(kp)
                else:
                    scat_wait(kp)

                @pl.when(j + NBUF - 1 < NCHUNK)
                def _():
                    dma_start(j + NBUF - 1, kp)

                dma_wait(j, k)
                scat_start(k)
                count_rows(k)

            if (NOUTER - 1) * NBUF + k >= NCHUNK:
                @pl.when(j < NCHUNK)
                def _():
                    step()
            else:
                step()
        return 0
    lax.fori_loop(0, NOUTER, outer, 0)
    scat_wait((NCHUNK - 1) % NBUF)

    plsc.subcore_barrier()

    out_base = c * SEG_PAD + s * SEG_PER_T

    def stage_step(t, _):
        src = s * SEG_PER_T + t * STAGE
        pltpu.sync_copy(acc.at[pl.ds(src, STAGE)], zbuf)
        pltpu.sync_copy(zbuf, sums_out.at[pl.ds(out_base + t * STAGE, STAGE)])
        return 0
    lax.fori_loop(0, NSTAGE, stage_step, 0)

    def repack(r, _):
        row = cnt1d[pl.ds(r * 16, 16)]
        xb0[lax.div(r, 8), pl.ds(lax.rem(r, 8) * 16, 16)] = row
        return 0
    lax.fori_loop(0, CR, repack, 0)
    pltpu.sync_copy(xb0, cnts_out.at[pl.ds(wid * HR, HR)])


_sc_sums = functools.partial(
    pl.kernel,
    out_type=(
        jax.ShapeDtypeStruct((NC * SEG_PAD, D), jnp.float32),
        jax.ShapeDtypeStruct((NW * HR, D), jnp.float32),
    ),
    mesh=plsc.VectorSubcoreMesh(core_axis_name="c", subcore_axis_name="s"),
    scratch_types=[
        pltpu.VMEM((CHUNK, D), jnp.float32),
        pltpu.VMEM((CHUNK, D), jnp.float32),
        pltpu.VMEM((CHUNK, D), jnp.float32),
        pltpu.VMEM((CHUNK,), jnp.int32),
        pltpu.VMEM((CHUNK,), jnp.int32),
        pltpu.VMEM((CHUNK,), jnp.int32),
        pltpu.VMEM((STAGE, D), jnp.float32),
        pltpu.VMEM((SEG_PAD,), jnp.float32),
        pltpu.VMEM((256,), jnp.float32),
        pltpu.VMEM_SHARED((SEG_PAD, D), jnp.float32),
        pltpu.SemaphoreType.DMA,
        pltpu.SemaphoreType.DMA,
        pltpu.SemaphoreType.DMA,
        pltpu.SemaphoreType.DMA,
        pltpu.SemaphoreType.DMA,
        pltpu.SemaphoreType.DMA,
        pltpu.SemaphoreType.DMA,
        pltpu.SemaphoreType.DMA,
        pltpu.SemaphoreType.DMA,
    ],
)(_sums_body)


BS = 1024


def _hsum_body(c_ref, o_ref):
    o_ref[...] = jnp.sum(c_ref[...], axis=0)


def _combine_body(s_ref, c_ref, o_ref):
    sums = s_ref[0] + s_ref[1]
    cnt = c_ref[...]
    o_ref[...] = sums / jnp.maximum(cnt, 1.0)


_EYE16 = jnp.asarray(np.eye(16, dtype=np.float32).reshape(256))


def kernel(x, batch):
    sums, cnts = _sc_sums(x, batch, _EYE16)
    sums = sums.reshape(NC, SEG_PAD, D)
    cnts = cnts.reshape(NW, HR, D)
    hsum = pl.pallas_call(
        _hsum_body,
        out_shape=jax.ShapeDtypeStruct((HR, D), jnp.float32),
        grid=(1,),
        in_specs=[pl.BlockSpec((NW, HR, D), lambda i: (0, 0, 0))],
        out_specs=pl.BlockSpec((HR, D), lambda i: (0, 0)),
    )(cnts)
    cnt_col = hsum.reshape(SEG_PAD, 1)
    out = pl.pallas_call(
        _combine_body,
        out_shape=jax.ShapeDtypeStruct((NUM_SEGMENTS, D), jnp.float32),
        grid=(SEG_PAD // BS,),
        in_specs=[
            pl.BlockSpec((NC, BS, D), lambda i: (0, i, 0)),
            pl.BlockSpec((BS, 1), lambda i: (i, 0)),
        ],
        out_specs=pl.BlockSpec((BS, D), lambda i: (i, 0)),
    )(sums, cnt_col)
    return out
